# baseline (speedup 1.0000x reference)
import functools

import jax
import jax.numpy as jnp
from jax import lax
from jax.experimental import pallas as pl
from jax.experimental.pallas import tpu as pltpu
from jax.experimental.pallas import tpu_sc as plsc

N = 10000
NP = 10240
IN_CH = 128
OUT_CH = 128
SPACE = 4
PROP = 64
K = 16
ROW_W = 128
TOPK_BLK = 256

NW = 32
NODES_PER_W = NP // NW
GRP = 8
N_GRP = NODES_PER_W // GRP


def _ka_body(x_ref, z_ref, ws_ref, bs_ref, wh_ref, bh_ref,
             table_ref, s_ref, sq_ref):
    xv = x_ref[...]
    dn = (((1,), (1,)), ((), ()))
    s = lax.dot_general(xv, ws_ref[...], dn,
                        preferred_element_type=jnp.float32) + bs_ref[...]
    h = lax.dot_general(xv, wh_ref[...], dn,
                        preferred_element_type=jnp.float32) + bh_ref[...]
    s_ref[...] = s
    sq_ref[...] = jnp.sum(s * s, axis=1, keepdims=True)
    pad = jnp.zeros((xv.shape[0], ROW_W - PROP - SPACE - 1), jnp.float32)
    table_ref[...] = jnp.concatenate([h, s, z_ref[...], pad], axis=1)


def _stage_a(xp, zp, W_s, b_s2, W_h, b_h2):
    return pl.pallas_call(
        _ka_body,
        out_shape=(
            jax.ShapeDtypeStruct((NP, ROW_W), jnp.float32),
            jax.ShapeDtypeStruct((NP, SPACE), jnp.float32),
            jax.ShapeDtypeStruct((NP, 1), jnp.float32),
        ),
    )(xp, zp, W_s, b_s2, W_h, b_h2)


def _kb_body(s_blk_ref, s_all_ref, sq_blk_ref, sq_row_ref, idx_ref):
    pid = pl.program_id(0)
    sr = s_blk_ref[...]
    sa = s_all_ref[...]
    dn = (((1,), (1,)), ((), ()))
    sqb = sq_blk_ref[...]
    row = pid * TOPK_BLK + lax.broadcasted_iota(
        jnp.int32, (TOPK_BLK, NP // 4), 0)
    inf = jnp.float32(jnp.inf)
    q = NP // 4
    vs = []
    cs = []
    for i in range(4):
        dq = (sqb + sq_row_ref[:, i * q:(i + 1) * q]) - 2.0 * lax.dot_general(
            sr, sa[i * q:(i + 1) * q, :], dn,
            preferred_element_type=jnp.float32)
        colq = i * q + lax.broadcasted_iota(
            jnp.int32, (TOPK_BLK, NP // 4), 1)
        m = colq == row
        if i == 3:
            m = m | (colq >= N)
        vs.append(jnp.where(m, inf, dq))
        cs.append(colq)
    for i, j in ((0, 1), (2, 3), (0, 2), (1, 3), (1, 2)):
        x, y = vs[i], vs[j]
        sw = y < x
        vs[i], vs[j] = jnp.where(sw, y, x), jnp.where(sw, x, y)
        cx, cy = cs[i], cs[j]
        cs[i], cs[j] = jnp.where(sw, cy, cx), jnp.where(sw, cx, cy)
    w, l1, l2, l3 = vs
    cw, c1, c2, c3 = cs
    bigi = jnp.int32(NP)
    for t in range(K):
        mn = jnp.min(w, axis=1, keepdims=True)
        am = jnp.min(jnp.where(w == mn, cw, bigi), axis=1, keepdims=True)
        idx_ref[:, t:t + 1] = am
        if t < K - 1:
            isp = cw == am
            w = jnp.where(isp, l1, w)
            cw = jnp.where(isp, c1, cw)
            l1 = jnp.where(isp, l2, l1)
            c1 = jnp.where(isp, c2, c1)
            l2 = jnp.where(isp, l3, l2)
            c2 = jnp.where(isp, c3, c2)
            l3 = jnp.where(isp, inf, l3)


def _stage_b(s, sq, sq_row):
    grid = NP // TOPK_BLK
    return pl.pallas_call(
        _kb_body,
        grid=(grid,),
        in_specs=[
            pl.BlockSpec((TOPK_BLK, SPACE), lambda i: (i, 0)),
            pl.BlockSpec((NP, SPACE), lambda i: (0, 0)),
            pl.BlockSpec((TOPK_BLK, 1), lambda i: (i, 0)),
            pl.BlockSpec((1, NP), lambda i: (0, 0)),
        ],
        out_specs=pl.BlockSpec((TOPK_BLK, K), lambda i: (i, 0)),
        out_shape=jax.ShapeDtypeStruct((NP, K), jnp.int32),
    )(s, s, sq, sq_row)


def _sc_body(table_hbm, idxf_hbm, out_hbm,
             idx_v, own_v, rows_v, stage_v, sem):
    wid = lax.axis_index("s") * 2 + lax.axis_index("c")
    base = wid * NODES_PER_W

    pltpu.sync_copy(idxf_hbm.at[pl.ds(base * K, NODES_PER_W * K)], idx_v)
    pltpu.sync_copy(table_hbm.at[pl.ds(base, NODES_PER_W)], own_v)

    lane = lax.iota(jnp.int32, 16)

    def group(g, _):
        pltpu.async_copy(
            table_hbm.at[idx_v.at[pl.ds(g * GRP * K, GRP * K)]],
            rows_v, sem).wait()
        for j in range(GRP):
            l = g * GRP + j
            ownvec = own_v[l, pl.ds(PROP, 16)]
            sm = lane < SPACE
            zm = lane == SPACE
            d_e = []
            dz_e = []
            for e in range(K):
                dv = rows_v[j * K + e, pl.ds(PROP, 16)] - ownvec
                d_e.append(jnp.sum(jnp.where(sm, dv * dv, 0.0)))
                dz_e.append(jnp.sum(jnp.where(zm, dv, 0.0)))
            sabs = jnp.abs(dz_e[0])
            for e in range(1, K):
                sabs = sabs + jnp.abs(dz_e[e])
            denv = jnp.full((16,), sabs + 1e-8, jnp.float32)
            zv = jnp.zeros((16,), jnp.float32)
            aM = [zv] * 4
            aS = [zv] * 4
            aD = [zv] * 4
            aX = [jnp.full((16,), -jnp.inf, jnp.float32)] * 4
            for e in range(K):
                we = jnp.exp(jnp.full((16,), d_e[e] * -10.0, jnp.float32))
                fe = jnp.full((16,), dz_e[e], jnp.float32) / denv
                ae = jnp.abs(fe)
                for c in range(4):
                    hv = rows_v[j * K + e, pl.ds(c * 16, 16)] * we
                    aM[c] = aM[c] + hv
                    aX[c] = jnp.maximum(aX[c], hv)
                    aS[c] = aS[c] + hv * ae
                    aD[c] = aD[c] + hv * fe
            inv_k = jnp.float32(1.0 / K)
            for c in range(4):
                stage_v[j, pl.ds(c * 16, 16)] = aM[c] * inv_k
                stage_v[j, pl.ds(64 + c * 16, 16)] = aX[c]
                stage_v[j, pl.ds(128 + c * 16, 16)] = aS[c] * inv_k
                stage_v[j, pl.ds(192 + c * 16, 16)] = aD[c]
        pltpu.sync_copy(stage_v, out_hbm.at[pl.ds(base + g * GRP, GRP)])
        return _

    lax.fori_loop(0, N_GRP, group, None)


def _stage_sc(table, idx_flat):
    mesh = plsc.VectorSubcoreMesh(core_axis_name="c", subcore_axis_name="s")
    f = pl.kernel(
        _sc_body,
        out_type=jax.ShapeDtypeStruct((NP, 4 * PROP), jnp.float32),
        mesh=mesh,
        compiler_params=pltpu.CompilerParams(needs_layout_passes=False),
        scratch_types=[
            pltpu.VMEM((NODES_PER_W * K,), jnp.int32),
            pltpu.VMEM((NODES_PER_W, ROW_W), jnp.float32),
            pltpu.VMEM((GRP * K, ROW_W), jnp.float32),
            pltpu.VMEM((GRP, 4 * PROP), jnp.float32),
            pltpu.SemaphoreType.DMA,
        ],
    )
    return f(table, idx_flat)


def _kc_body(x_ref, agg_ref, wskip_ref, wagg_ref, bagg_ref,
             g_ref, b_ref, out_ref):
    dn = (((1,), (1,)), ((), ()))
    o = (lax.dot_general(x_ref[...], wskip_ref[...], dn,
                         preferred_element_type=jnp.float32)
         + lax.dot_general(agg_ref[...], wagg_ref[...], dn,
                           preferred_element_type=jnp.float32)
         + bagg_ref[...])
    mask = lax.broadcasted_iota(jnp.int32, (NP, OUT_CH), 0) < N
    om = jnp.where(mask, o, 0.0)
    mu = jnp.sum(om, axis=0, keepdims=True) / N
    dif = o - mu
    dm = jnp.where(mask, dif, 0.0)
    var = jnp.sum(dm * dm, axis=0, keepdims=True) / N
    out_ref[...] = jnp.maximum(
        dif / jnp.sqrt(var + 1e-5) * g_ref[...] + b_ref[...], 0.0)


def _stage_c(xp, agg, W_skip, W_agg, b_agg2, gamma2, beta2):
    return pl.pallas_call(
        _kc_body,
        out_shape=jax.ShapeDtypeStruct((NP, OUT_CH), jnp.float32),
    )(xp, agg, W_skip, W_agg, b_agg2, gamma2, beta2)


def kernel(x, z, batch, W_s, b_s, W_h, b_h, W_skip, W_agg, b_agg,
           gamma, beta):
    del batch
    xp = jnp.pad(x, ((0, NP - N), (0, 0)))
    zp = jnp.pad(z, (0, NP - N)).reshape(NP, 1)
    table, s, sq = _stage_a(xp, zp, W_s, b_s.reshape(1, SPACE),
                            W_h, b_h.reshape(1, PROP))
    idx = _stage_b(s, sq, sq.reshape(1, NP))
    agg = _stage_sc(table, idx.reshape(-1))
    out = _stage_c(xp, agg, W_skip, W_agg, b_agg.reshape(1, OUT_CH),
                   gamma.reshape(1, OUT_CH), beta.reshape(1, OUT_CH))
    return out[:N]


# speedup vs baseline: 1.2537x; 1.2537x over previous
import functools

import jax
import jax.numpy as jnp
from jax import lax
from jax.experimental import pallas as pl
from jax.experimental.pallas import tpu as pltpu
from jax.experimental.pallas import tpu_sc as plsc

N = 10000
NP = 10240
IN_CH = 128
OUT_CH = 128
SPACE = 4
PROP = 64
K = 16
ROW_W = 128
TOPK_BLK = 128

NW = 32
HALFN = NP // 2
NODES_PER_W = HALFN // NW
GRP = 8
N_GRP = NODES_PER_W // GRP


def _ka_body(x_ref, z_ref, ws_ref, bs_ref, wh_ref, bh_ref,
             table_ref, s_ref, sq_ref):
    xv = x_ref[...]
    dn = (((1,), (1,)), ((), ()))
    s = lax.dot_general(xv, ws_ref[...], dn,
                        preferred_element_type=jnp.float32) + bs_ref[...]
    h = lax.dot_general(xv, wh_ref[...], dn,
                        preferred_element_type=jnp.float32) + bh_ref[...]
    s_ref[...] = s
    sq_ref[...] = jnp.sum(s * s, axis=1, keepdims=True)
    pad = jnp.zeros((xv.shape[0], ROW_W - PROP - SPACE - 1), jnp.float32)
    table_ref[...] = jnp.concatenate([h, s, z_ref[...], pad], axis=1)


def _stage_a(xp, zp, W_s, b_s2, W_h, b_h2):
    return pl.pallas_call(
        _ka_body,
        out_shape=(
            jax.ShapeDtypeStruct((NP, ROW_W), jnp.float32),
            jax.ShapeDtypeStruct((NP, SPACE), jnp.float32),
            jax.ShapeDtypeStruct((NP, 1), jnp.float32),
        ),
    )(xp, zp, W_s, b_s2, W_h, b_h2)


def _kb_body(s_blk_ref, s_all_ref, sq_blk_ref, sq_row_ref, idx_ref, *, roff):
    pid = pl.program_id(0)
    sr = s_blk_ref[...]
    sa = s_all_ref[...]
    dn = (((1,), (1,)), ((), ()))
    sqb = sq_blk_ref[...]
    row = roff + pid * TOPK_BLK + lax.broadcasted_iota(
        jnp.int32, (TOPK_BLK, NP // 4), 0)
    inf = jnp.float32(jnp.inf)
    q = NP // 4
    vs = []
    cs = []
    for i in range(4):
        dq = (sqb + sq_row_ref[:, i * q:(i + 1) * q]) - 2.0 * lax.dot_general(
            sr, sa[i * q:(i + 1) * q, :], dn,
            preferred_element_type=jnp.float32)
        colq = i * q + lax.broadcasted_iota(
            jnp.int32, (TOPK_BLK, NP // 4), 1)
        m = colq == row
        if i == 3:
            m = m | (colq >= N)
        vs.append(jnp.where(m, inf, dq))
        cs.append(colq)
    for i, j in ((0, 1), (2, 3), (0, 2), (1, 3), (1, 2)):
        x, y = vs[i], vs[j]
        sw = y < x
        vs[i], vs[j] = jnp.where(sw, y, x), jnp.where(sw, x, y)
        cx, cy = cs[i], cs[j]
        cs[i], cs[j] = jnp.where(sw, cy, cx), jnp.where(sw, cx, cy)
    w, l1, l2, l3 = vs
    cw, c1, c2, c3 = cs
    bigi = jnp.int32(NP)
    for t in range(K):
        mn = jnp.min(w, axis=1, keepdims=True)
        am = jnp.min(jnp.where(w == mn, cw, bigi), axis=1, keepdims=True)
        idx_ref[:, t:t + 1] = am
        if t < K - 1:
            isp = cw == am
            w = jnp.where(isp, l1, w)
            cw = jnp.where(isp, c1, cw)
            l1 = jnp.where(isp, l2, l1)
            c1 = jnp.where(isp, c2, c1)
            l2 = jnp.where(isp, l3, l2)
            c2 = jnp.where(isp, c3, c2)
            l3 = jnp.where(isp, inf, l3)


def _stage_b(s, sq, sq_row, roff):
    grid = HALFN // TOPK_BLK
    boff = roff // TOPK_BLK
    return pl.pallas_call(
        functools.partial(_kb_body, roff=roff),
        grid=(grid,),
        in_specs=[
            pl.BlockSpec((TOPK_BLK, SPACE), lambda i: (i + boff, 0)),
            pl.BlockSpec((NP, SPACE), lambda i: (0, 0)),
            pl.BlockSpec((TOPK_BLK, 1), lambda i: (i + boff, 0)),
            pl.BlockSpec((1, NP), lambda i: (0, 0)),
        ],
        out_specs=pl.BlockSpec((TOPK_BLK, K), lambda i: (i, 0)),
        out_shape=jax.ShapeDtypeStruct((HALFN, K), jnp.int32),
    )(s, s, sq, sq_row)


def _sc_body(table_hbm, idxf_hbm, out_hbm,
             idx_v, own_v, rows_v, stage_v, sem, *, roff):
    wid = lax.axis_index("s") * 2 + lax.axis_index("c")
    base = wid * NODES_PER_W

    pltpu.sync_copy(idxf_hbm.at[pl.ds(base * K, NODES_PER_W * K)], idx_v)
    pltpu.sync_copy(table_hbm.at[pl.ds(roff + base, NODES_PER_W)], own_v)

    lane = lax.iota(jnp.int32, 16)

    def group(g, _):
        pltpu.async_copy(
            table_hbm.at[idx_v.at[pl.ds(g * GRP * K, GRP * K)]],
            rows_v, sem).wait()
        for j in range(GRP):
            l = g * GRP + j
            ownvec = own_v[l, pl.ds(PROP, 16)]
            sm = lane < SPACE
            zm = lane == SPACE
            d_e = []
            dz_e = []
            for e in range(K):
                dv = rows_v[j * K + e, pl.ds(PROP, 16)] - ownvec
                d_e.append(jnp.sum(jnp.where(sm, dv * dv, 0.0)))
                dz_e.append(jnp.sum(jnp.where(zm, dv, 0.0)))
            sabs = jnp.abs(dz_e[0])
            for e in range(1, K):
                sabs = sabs + jnp.abs(dz_e[e])
            denv = jnp.full((16,), sabs + 1e-8, jnp.float32)
            zv = jnp.zeros((16,), jnp.float32)
            aM = [zv] * 4
            aS = [zv] * 4
            aD = [zv] * 4
            aX = [jnp.full((16,), -jnp.inf, jnp.float32)] * 4
            for e in range(K):
                we = jnp.exp(jnp.full((16,), d_e[e] * -10.0, jnp.float32))
                fe = jnp.full((16,), dz_e[e], jnp.float32) / denv
                ae = jnp.abs(fe)
                for c in range(4):
                    hv = rows_v[j * K + e, pl.ds(c * 16, 16)] * we
                    aM[c] = aM[c] + hv
                    aX[c] = jnp.maximum(aX[c], hv)
                    aS[c] = aS[c] + hv * ae
                    aD[c] = aD[c] + hv * fe
            inv_k = jnp.float32(1.0 / K)
            for c in range(4):
                stage_v[j, pl.ds(c * 16, 16)] = aM[c] * inv_k
                stage_v[j, pl.ds(64 + c * 16, 16)] = aX[c]
                stage_v[j, pl.ds(128 + c * 16, 16)] = aS[c] * inv_k
                stage_v[j, pl.ds(192 + c * 16, 16)] = aD[c]
        pltpu.sync_copy(stage_v, out_hbm.at[pl.ds(base + g * GRP, GRP)])
        return _

    lax.fori_loop(0, N_GRP, group, None)


def _stage_sc(table, idx_flat, roff):
    mesh = plsc.VectorSubcoreMesh(core_axis_name="c", subcore_axis_name="s")
    f = pl.kernel(
        functools.partial(_sc_body, roff=roff),
        out_type=jax.ShapeDtypeStruct((HALFN, 4 * PROP), jnp.float32),
        mesh=mesh,
        compiler_params=pltpu.CompilerParams(needs_layout_passes=False),
        scratch_types=[
            pltpu.VMEM((NODES_PER_W * K,), jnp.int32),
            pltpu.VMEM((NODES_PER_W, ROW_W), jnp.float32),
            pltpu.VMEM((GRP * K, ROW_W), jnp.float32),
            pltpu.VMEM((GRP, 4 * PROP), jnp.float32),
            pltpu.SemaphoreType.DMA,
        ],
    )
    return f(table, idx_flat)


def _kc_body(x_ref, agg_ref, wskip_ref, wagg_ref, bagg_ref,
             g_ref, b_ref, out_ref):
    dn = (((1,), (1,)), ((), ()))
    o = (lax.dot_general(x_ref[...], wskip_ref[...], dn,
                         preferred_element_type=jnp.float32)
         + lax.dot_general(agg_ref[...], wagg_ref[...], dn,
                           preferred_element_type=jnp.float32)
         + bagg_ref[...])
    mask = lax.broadcasted_iota(jnp.int32, (NP, OUT_CH), 0) < N
    om = jnp.where(mask, o, 0.0)
    mu = jnp.sum(om, axis=0, keepdims=True) / N
    dif = o - mu
    dm = jnp.where(mask, dif, 0.0)
    var = jnp.sum(dm * dm, axis=0, keepdims=True) / N
    out_ref[...] = jnp.maximum(
        dif / jnp.sqrt(var + 1e-5) * g_ref[...] + b_ref[...], 0.0)


def _stage_c(xp, agg, W_skip, W_agg, b_agg2, gamma2, beta2):
    return pl.pallas_call(
        _kc_body,
        out_shape=jax.ShapeDtypeStruct((NP, OUT_CH), jnp.float32),
    )(xp, agg, W_skip, W_agg, b_agg2, gamma2, beta2)


def kernel(x, z, batch, W_s, b_s, W_h, b_h, W_skip, W_agg, b_agg,
           gamma, beta):
    del batch
    xp = jnp.pad(x, ((0, NP - N), (0, 0)))
    zp = jnp.pad(z, (0, NP - N)).reshape(NP, 1)
    table, s, sq = _stage_a(xp, zp, W_s, b_s.reshape(1, SPACE),
                            W_h, b_h.reshape(1, PROP))
    sq_row = sq.reshape(1, NP)
    idx1 = _stage_b(s, sq, sq_row, 0)
    agg1 = _stage_sc(table, idx1.reshape(-1), 0)
    idx2 = _stage_b(s, sq, sq_row, HALFN)
    agg2 = _stage_sc(table, idx2.reshape(-1), HALFN)
    agg = jnp.concatenate([agg1, agg2], axis=0)
    out = _stage_c(xp, agg, W_skip, W_agg, b_agg.reshape(1, OUT_CH),
                   gamma.reshape(1, OUT_CH), beta.reshape(1, OUT_CH))
    return out[:N]


# speedup vs baseline: 1.2617x; 1.0064x over previous
import functools

import jax
import jax.numpy as jnp
from jax import lax
from jax.experimental import pallas as pl
from jax.experimental.pallas import tpu as pltpu
from jax.experimental.pallas import tpu_sc as plsc

N = 10000
NP = 10240
IN_CH = 128
OUT_CH = 128
SPACE = 4
PROP = 64
K = 16
ROW_W = 128
TOPK_BLK = 128

NW = 32
NSPLIT = 4
HALFN = NP // NSPLIT
NODES_PER_W = HALFN // NW
GRP = 8
N_GRP = NODES_PER_W // GRP


def _ka_body(x_ref, z_ref, ws_ref, bs_ref, wh_ref, bh_ref,
             table_ref, s_ref, sq_ref):
    xv = x_ref[...]
    dn = (((1,), (1,)), ((), ()))
    s = lax.dot_general(xv, ws_ref[...], dn,
                        preferred_element_type=jnp.float32) + bs_ref[...]
    h = lax.dot_general(xv, wh_ref[...], dn,
                        preferred_element_type=jnp.float32) + bh_ref[...]
    s_ref[...] = s
    sq_ref[...] = jnp.sum(s * s, axis=1, keepdims=True)
    pad = jnp.zeros((xv.shape[0], ROW_W - PROP - SPACE - 1), jnp.float32)
    table_ref[...] = jnp.concatenate([h, s, z_ref[...], pad], axis=1)


def _stage_a(xp, zp, W_s, b_s2, W_h, b_h2):
    return pl.pallas_call(
        _ka_body,
        out_shape=(
            jax.ShapeDtypeStruct((NP, ROW_W), jnp.float32),
            jax.ShapeDtypeStruct((NP, SPACE), jnp.float32),
            jax.ShapeDtypeStruct((NP, 1), jnp.float32),
        ),
    )(xp, zp, W_s, b_s2, W_h, b_h2)


def _kb_body(s_blk_ref, s_all_ref, sq_blk_ref, sq_row_ref, idx_ref, *, roff):
    pid = pl.program_id(0)
    sr = s_blk_ref[...]
    sa = s_all_ref[...]
    dn = (((1,), (1,)), ((), ()))
    sqb = sq_blk_ref[...]
    row = roff + pid * TOPK_BLK + lax.broadcasted_iota(
        jnp.int32, (TOPK_BLK, NP // 4), 0)
    inf = jnp.float32(jnp.inf)
    q = NP // 4
    vs = []
    cs = []
    for i in range(4):
        dq = (sqb + sq_row_ref[:, i * q:(i + 1) * q]) - 2.0 * lax.dot_general(
            sr, sa[i * q:(i + 1) * q, :], dn,
            preferred_element_type=jnp.float32)
        colq = i * q + lax.broadcasted_iota(
            jnp.int32, (TOPK_BLK, NP // 4), 1)
        m = colq == row
        if i == 3:
            m = m | (colq >= N)
        vs.append(jnp.where(m, inf, dq))
        cs.append(colq)
    for i, j in ((0, 1), (2, 3), (0, 2), (1, 3), (1, 2)):
        x, y = vs[i], vs[j]
        sw = y < x
        vs[i], vs[j] = jnp.where(sw, y, x), jnp.where(sw, x, y)
        cx, cy = cs[i], cs[j]
        cs[i], cs[j] = jnp.where(sw, cy, cx), jnp.where(sw, cx, cy)
    w, l1, l2, l3 = vs
    cw, c1, c2, c3 = cs
    bigi = jnp.int32(NP)
    for t in range(K):
        mn = jnp.min(w, axis=1, keepdims=True)
        am = jnp.min(jnp.where(w == mn, cw, bigi), axis=1, keepdims=True)
        idx_ref[:, t:t + 1] = am
        if t < K - 1:
            isp = cw == am
            w = jnp.where(isp, l1, w)
            cw = jnp.where(isp, c1, cw)
            l1 = jnp.where(isp, l2, l1)
            c1 = jnp.where(isp, c2, c1)
            l2 = jnp.where(isp, l3, l2)
            c2 = jnp.where(isp, c3, c2)
            l3 = jnp.where(isp, inf, l3)


def _stage_b(s, sq, sq_row, roff):
    grid = HALFN // TOPK_BLK
    boff = roff // TOPK_BLK
    return pl.pallas_call(
        functools.partial(_kb_body, roff=roff),
        grid=(grid,),
        in_specs=[
            pl.BlockSpec((TOPK_BLK, SPACE), lambda i: (i + boff, 0)),
            pl.BlockSpec((NP, SPACE), lambda i: (0, 0)),
            pl.BlockSpec((TOPK_BLK, 1), lambda i: (i + boff, 0)),
            pl.BlockSpec((1, NP), lambda i: (0, 0)),
        ],
        out_specs=pl.BlockSpec((TOPK_BLK, K), lambda i: (i, 0)),
        out_shape=jax.ShapeDtypeStruct((HALFN, K), jnp.int32),
    )(s, s, sq, sq_row)


def _sc_body(table_hbm, idxf_hbm, out_hbm,
             idx_v, own_v, rows_v, stage_v, sem, *, roff):
    wid = lax.axis_index("s") * 2 + lax.axis_index("c")
    base = wid * NODES_PER_W

    pltpu.sync_copy(idxf_hbm.at[pl.ds(base * K, NODES_PER_W * K)], idx_v)
    pltpu.sync_copy(table_hbm.at[pl.ds(roff + base, NODES_PER_W)], own_v)

    lane = lax.iota(jnp.int32, 16)

    def group(g, _):
        pltpu.async_copy(
            table_hbm.at[idx_v.at[pl.ds(g * GRP * K, GRP * K)]],
            rows_v, sem).wait()
        for j in range(GRP):
            l = g * GRP + j
            ownvec = own_v[l, pl.ds(PROP, 16)]
            sm = lane < SPACE
            zm = lane == SPACE
            d_e = []
            dz_e = []
            for e in range(K):
                dv = rows_v[j * K + e, pl.ds(PROP, 16)] - ownvec
                d_e.append(jnp.sum(jnp.where(sm, dv * dv, 0.0)))
                dz_e.append(jnp.sum(jnp.where(zm, dv, 0.0)))
            sabs = jnp.abs(dz_e[0])
            for e in range(1, K):
                sabs = sabs + jnp.abs(dz_e[e])
            denv = jnp.full((16,), sabs + 1e-8, jnp.float32)
            zv = jnp.zeros((16,), jnp.float32)
            aM = [zv] * 4
            aS = [zv] * 4
            aD = [zv] * 4
            aX = [jnp.full((16,), -jnp.inf, jnp.float32)] * 4
            for e in range(K):
                we = jnp.exp(jnp.full((16,), d_e[e] * -10.0, jnp.float32))
                fe = jnp.full((16,), dz_e[e], jnp.float32) / denv
                ae = jnp.abs(fe)
                for c in range(4):
                    hv = rows_v[j * K + e, pl.ds(c * 16, 16)] * we
                    aM[c] = aM[c] + hv
                    aX[c] = jnp.maximum(aX[c], hv)
                    aS[c] = aS[c] + hv * ae
                    aD[c] = aD[c] + hv * fe
            inv_k = jnp.float32(1.0 / K)
            for c in range(4):
                stage_v[j, pl.ds(c * 16, 16)] = aM[c] * inv_k
                stage_v[j, pl.ds(64 + c * 16, 16)] = aX[c]
                stage_v[j, pl.ds(128 + c * 16, 16)] = aS[c] * inv_k
                stage_v[j, pl.ds(192 + c * 16, 16)] = aD[c]
        pltpu.sync_copy(stage_v, out_hbm.at[pl.ds(base + g * GRP, GRP)])
        return _

    lax.fori_loop(0, N_GRP, group, None)


def _stage_sc(table, idx_flat, roff):
    mesh = plsc.VectorSubcoreMesh(core_axis_name="c", subcore_axis_name="s")
    f = pl.kernel(
        functools.partial(_sc_body, roff=roff),
        out_type=jax.ShapeDtypeStruct((HALFN, 4 * PROP), jnp.float32),
        mesh=mesh,
        compiler_params=pltpu.CompilerParams(needs_layout_passes=False),
        scratch_types=[
            pltpu.VMEM((NODES_PER_W * K,), jnp.int32),
            pltpu.VMEM((NODES_PER_W, ROW_W), jnp.float32),
            pltpu.VMEM((GRP * K, ROW_W), jnp.float32),
            pltpu.VMEM((GRP, 4 * PROP), jnp.float32),
            pltpu.SemaphoreType.DMA,
        ],
    )
    return f(table, idx_flat)


def _kc_body(x_ref, agg_ref, wskip_ref, wagg_ref, bagg_ref,
             g_ref, b_ref, out_ref):
    dn = (((1,), (1,)), ((), ()))
    o = (lax.dot_general(x_ref[...], wskip_ref[...], dn,
                         preferred_element_type=jnp.float32)
         + lax.dot_general(agg_ref[...], wagg_ref[...], dn,
                           preferred_element_type=jnp.float32)
         + bagg_ref[...])
    mask = lax.broadcasted_iota(jnp.int32, (NP, OUT_CH), 0) < N
    om = jnp.where(mask, o, 0.0)
    mu = jnp.sum(om, axis=0, keepdims=True) / N
    dif = o - mu
    dm = jnp.where(mask, dif, 0.0)
    var = jnp.sum(dm * dm, axis=0, keepdims=True) / N
    out_ref[...] = jnp.maximum(
        dif / jnp.sqrt(var + 1e-5) * g_ref[...] + b_ref[...], 0.0)


def _stage_c(xp, agg, W_skip, W_agg, b_agg2, gamma2, beta2):
    return pl.pallas_call(
        _kc_body,
        out_shape=jax.ShapeDtypeStruct((NP, OUT_CH), jnp.float32),
    )(xp, agg, W_skip, W_agg, b_agg2, gamma2, beta2)


def kernel(x, z, batch, W_s, b_s, W_h, b_h, W_skip, W_agg, b_agg,
           gamma, beta):
    del batch
    xp = jnp.pad(x, ((0, NP - N), (0, 0)))
    zp = jnp.pad(z, (0, NP - N)).reshape(NP, 1)
    table, s, sq = _stage_a(xp, zp, W_s, b_s.reshape(1, SPACE),
                            W_h, b_h.reshape(1, PROP))
    sq_row = sq.reshape(1, NP)
    aggs = []
    for p in range(NSPLIT):
        idx_p = _stage_b(s, sq, sq_row, p * HALFN)
        aggs.append(_stage_sc(table, idx_p.reshape(-1), p * HALFN))
    agg = jnp.concatenate(aggs, axis=0)
    out = _stage_c(xp, agg, W_skip, W_agg, b_agg.reshape(1, OUT_CH),
                   gamma.reshape(1, OUT_CH), beta.reshape(1, OUT_CH))
    return out[:N]
